# baseline (speedup 1.0000x reference)
import functools

import jax
import jax.numpy as jnp
from jax.experimental import pallas as pl
from jax.experimental.pallas import tpu as pltpu

_B = 32
_K = 64
_OUT = 127
_DX = 1e-06


def _sv_kernel(oi8_ref, ojb_ref, ojr_ref, x_ref, w_ref, z_ref, sens_ref,
               acc_e, acc_o, dma_sem, *, nb):
    h = pl.program_id(0)
    p = pl.program_id(1)

    @pl.when(p == 0)
    def _init():
        cp = pltpu.make_async_copy(z_ref.at[pl.ds(h * nb, nb)], sens_ref,
                                   dma_sem)
        cp.start()
        cp.wait()

    w = w_ref[0]
    zr = jnp.zeros((1, _K), w.dtype)
    wr = jnp.concatenate([zr, w, zr], axis=0)
    zc = jnp.zeros((_K + 2, 1), w.dtype)
    wp = jnp.concatenate([zc, wr, zc], axis=1)
    rev = (wp[0:65, 0:65] + wp[1:66, 0:65]
           + wp[0:65, 1:66] + wp[1:66, 1:66])

    rows = []
    for J in range(_B):
        pieces = []
        if J > 0:
            pieces.append(jnp.zeros((65, 2 * J), jnp.float32))
        pieces.append(rev)
        pieces.append(jnp.zeros((65, 128 - 65 - 2 * J), jnp.float32))
        rows.append(jnp.concatenate(pieces, axis=1)[None])
    S2 = jnp.concatenate(rows, axis=0).reshape(_B, 65 * 128)

    Xm = x_ref[:, 0].reshape(nb * _B, _B)
    R = jnp.dot(Xm, S2, preferred_element_type=jnp.float32)

    acc_e[...] = jnp.zeros_like(acc_e)
    acc_o[...] = jnp.zeros_like(acc_o)
    for k in range(33):
        slab = R[:, (64 - 2 * k) * 128:(64 - 2 * k) * 128 + _OUT]
        acc_e[:, k:k + _B, :] = acc_e[:, k:k + _B, :] + slab.reshape(nb, _B, _OUT)
    for k in range(32):
        slab = R[:, (63 - 2 * k) * 128:(63 - 2 * k) * 128 + _OUT]
        acc_o[:, k:k + _B, :] = acc_o[:, k:k + _B, :] + slab.reshape(nb, _B, _OUT)

    e = acc_e[...]
    o = acc_o[...]
    o_shift = jnp.concatenate(
        [jnp.zeros((nb, 1, _OUT), jnp.float32), o[:, :63, :]], axis=1)
    ppad = jnp.stack([o_shift, e], axis=2).reshape(nb, 128, _OUT)

    wide = jnp.concatenate(
        [ppad, jnp.zeros((nb, 128, 256 - _OUT), jnp.float32)], axis=2)
    rolled = pltpu.roll(wide, ojr_ref[p], axis=2)

    r0 = oi8_ref[p] * 8
    c0 = ojb_ref[p] * 128
    cur = sens_ref[:, pl.ds(r0, 128), pl.ds(c0, 256)]
    sens_ref[:, pl.ds(r0, 128), pl.ds(c0, 256)] = cur + rolled


def kernel(imgs, psf_stack, X, Y, x_centers, y_centers):
    N, H, W = imgs.shape
    P = psf_stack.shape[0]
    X_N, Y_N = X.shape
    nBH, nBW = H // _B, W // _B

    x = imgs.reshape(N, nBH, _B, nBW, _B)
    x = jnp.transpose(x, (0, 2, 4, 3, 1)).reshape(N, _B, _B, nBW * nBH)
    blocks = jnp.transpose(x, (0, 3, 1, 2))

    ic = X_N // 2 + jnp.round(x_centers / _DX).astype(jnp.int32)
    jc = Y_N // 2 + jnp.round(y_centers / _DX).astype(jnp.int32)
    hk = _OUT // 2
    oi = jnp.clip(ic - hk, 0, X_N - _OUT).astype(jnp.int32)
    oj = jnp.clip(jc - hk, 0, Y_N - _OUT).astype(jnp.int32)
    oi8 = jnp.clip((oi - 1) // 8, 0, (X_N - 128) // 8).astype(jnp.int32)
    ojb = jnp.clip(oj // 128, 0, (Y_N - 256) // 128).astype(jnp.int32)
    ojr = (oj - ojb * 128).astype(jnp.int32)

    psf_f = psf_stack[:, :, ::-1]

    half = 2 if N % 2 == 0 else 1
    nb = N // half

    out = pl.pallas_call(
        functools.partial(_sv_kernel, nb=nb),
        out_shape=jax.ShapeDtypeStruct((N, X_N, Y_N), jnp.float32),
        grid_spec=pltpu.PrefetchScalarGridSpec(
            num_scalar_prefetch=3,
            grid=(half, P),
            in_specs=[
                pl.BlockSpec((nb, 1, _B, _B), lambda h, p, a, b, c: (h, p, 0, 0)),
                pl.BlockSpec((1, _K, _K), lambda h, p, a, b, c: (p, 0, 0)),
                pl.BlockSpec(memory_space=pltpu.MemorySpace.HBM),
            ],
            out_specs=pl.BlockSpec((nb, X_N, Y_N), lambda h, p, a, b, c: (h, 0, 0)),
            scratch_shapes=[
                pltpu.VMEM((nb, 64, _OUT), jnp.float32),
                pltpu.VMEM((nb, 64, _OUT), jnp.float32),
                pltpu.SemaphoreType.DMA,
            ],
        ),
        compiler_params=pltpu.CompilerParams(
            dimension_semantics=("parallel", "arbitrary"),
            vmem_limit_bytes=56 * 1024 * 1024,
        ),
        name="sv_convolution",
    )(oi8, ojb, ojr, blocks, psf_f,
      jnp.zeros((N, X_N, Y_N), jnp.float32))
    return out


# speedup vs baseline: 1.1377x; 1.1377x over previous
import functools

import jax
import jax.numpy as jnp
from jax.experimental import pallas as pl
from jax.experimental.pallas import tpu as pltpu

_B = 32
_K = 64
_OUT = 127
_DX = 1e-06


def _sv_kernel(oi8_ref, ojb_ref, ojr_ref, x_ref, w_ref, sens_ref,
               acc_e, acc_o, *, nb):
    p = pl.program_id(1)

    @pl.when(p == 0)
    def _init():
        sens_ref[...] = jnp.zeros_like(sens_ref)

    w = w_ref[0]
    zr = jnp.zeros((1, _K), w.dtype)
    wr = jnp.concatenate([zr, w, zr], axis=0)
    zc = jnp.zeros((_K + 2, 1), w.dtype)
    wp = jnp.concatenate([zc, wr, zc], axis=1)
    rev = (wp[0:65, 0:65] + wp[1:66, 0:65]
           + wp[0:65, 1:66] + wp[1:66, 1:66]).astype(jnp.bfloat16)

    rows = []
    for J in range(_B):
        pieces = []
        if J > 0:
            pieces.append(jnp.zeros((65, 2 * J), jnp.bfloat16))
        pieces.append(rev)
        pieces.append(jnp.zeros((65, 128 - 65 - 2 * J), jnp.bfloat16))
        rows.append(jnp.concatenate(pieces, axis=1)[None])
    S2 = jnp.concatenate(rows, axis=0).reshape(_B, 65 * 128)

    Xm = x_ref[:, 0].reshape(nb * _B, _B)
    R = jnp.dot(Xm, S2, preferred_element_type=jnp.float32)

    acc_e[...] = jnp.zeros_like(acc_e)
    acc_o[...] = jnp.zeros_like(acc_o)
    for k in range(33):
        slab = R[:, (64 - 2 * k) * 128:(64 - 2 * k) * 128 + _OUT]
        acc_e[:, k:k + _B, :] = acc_e[:, k:k + _B, :] + slab.reshape(nb, _B, _OUT)
    for k in range(32):
        slab = R[:, (63 - 2 * k) * 128:(63 - 2 * k) * 128 + _OUT]
        acc_o[:, k:k + _B, :] = acc_o[:, k:k + _B, :] + slab.reshape(nb, _B, _OUT)

    e = acc_e[...]
    o = acc_o[...]
    o_shift = jnp.concatenate(
        [jnp.zeros((nb, 1, _OUT), jnp.float32), o[:, :63, :]], axis=1)
    ppad = jnp.stack([o_shift, e], axis=2).reshape(nb, 128, _OUT)

    wide = jnp.concatenate(
        [ppad, jnp.zeros((nb, 128, 256 - _OUT), jnp.float32)], axis=2)
    rolled = pltpu.roll(wide, ojr_ref[p], axis=2)

    r0 = oi8_ref[p] * 8
    c0 = ojb_ref[p] * 128
    cur = sens_ref[:, pl.ds(r0, 128), pl.ds(c0, 256)]
    sens_ref[:, pl.ds(r0, 128), pl.ds(c0, 256)] = cur + rolled


def kernel(imgs, psf_stack, X, Y, x_centers, y_centers):
    N, H, W = imgs.shape
    P = psf_stack.shape[0]
    X_N, Y_N = X.shape
    nBH, nBW = H // _B, W // _B

    x = imgs.reshape(N, nBH, _B, nBW, _B)
    x = jnp.transpose(x, (0, 2, 4, 3, 1)).reshape(N, _B, _B, nBW * nBH)
    blocks = jnp.transpose(x, (0, 3, 1, 2)).astype(jnp.bfloat16)

    ic = X_N // 2 + jnp.round(x_centers / _DX).astype(jnp.int32)
    jc = Y_N // 2 + jnp.round(y_centers / _DX).astype(jnp.int32)
    hk = _OUT // 2
    oi = jnp.clip(ic - hk, 0, X_N - _OUT).astype(jnp.int32)
    oj = jnp.clip(jc - hk, 0, Y_N - _OUT).astype(jnp.int32)
    oi8 = jnp.clip((oi - 1) // 8, 0, (X_N - 128) // 8).astype(jnp.int32)
    ojb = jnp.clip(oj // 128, 0, (Y_N - 256) // 128).astype(jnp.int32)
    ojr = (oj - ojb * 128).astype(jnp.int32)

    psf_f = psf_stack[:, :, ::-1]

    half = 2 if N % 2 == 0 else 1
    nb = N // half

    out = pl.pallas_call(
        functools.partial(_sv_kernel, nb=nb),
        out_shape=jax.ShapeDtypeStruct((N, X_N, Y_N), jnp.float32),
        grid_spec=pltpu.PrefetchScalarGridSpec(
            num_scalar_prefetch=3,
            grid=(half, P),
            in_specs=[
                pl.BlockSpec((nb, 1, _B, _B), lambda h, p, a, b, c: (h, p, 0, 0)),
                pl.BlockSpec((1, _K, _K), lambda h, p, a, b, c: (p, 0, 0)),
            ],
            out_specs=pl.BlockSpec((nb, X_N, Y_N), lambda h, p, a, b, c: (h, 0, 0)),
            scratch_shapes=[
                pltpu.VMEM((nb, 64, _OUT), jnp.float32),
                pltpu.VMEM((nb, 64, _OUT), jnp.float32),
            ],
        ),
        compiler_params=pltpu.CompilerParams(
            dimension_semantics=("parallel", "arbitrary"),
            vmem_limit_bytes=56 * 1024 * 1024,
        ),
        name="sv_convolution",
    )(oi8, ojb, ojr, blocks, psf_f)
    return out
